# Initial kernel scaffold; baseline (speedup 1.0000x reference)
import jax
import jax.numpy as jnp
from jax.experimental import pallas as pl


def kernel(SH, aatype_probs, Rmats, tpos, W_e, b_e, W1, b1, W2, b2, W_t, b_t, W_s, b_s, template_local, template_exists, tors_axis, aff_idx, aff_mask, G_counts, scale_mask, scale_parent):
    raise NotImplementedError("write your pallas kernel here")


# speedup vs baseline: 135.2514x; 135.2514x over previous
import jax
import jax.numpy as jnp
from jax.experimental import pallas as pl
from functools import partial

_T = 512
_PAD = 16

_R_TX, _R_TY, _R_TZ = 0, 16, 32
_R_UW = 48
_R_VW = 112
_R_AF = 176
_R_SP = 240
_R_SM = 256
_R_AX = 272
_ROWS = 288


def _body(sh_ref, pr_ref, rt_ref, we_ref, be_ref, w1h_ref, w1p_ref, b1_ref,
          w2_ref, b2_ref, wh_ref, bh_ref, meta_ref, out_ref):
    f32 = jnp.float32
    sh = sh_ref[...]
    pr = pr_ref[...]
    rt = rt_ref[...]

    dot = partial(jax.lax.dot_general, preferred_element_type=f32)
    mm = lambda a, b: dot(a, b, (((1,), (0,)), ((), ())))

    h = mm(we_ref[...], sh) + be_ref[...]
    z = mm(w1h_ref[...], h) + mm(w1p_ref[...], pr) + b1_ref[...]
    z = z * (1.0 / (1.0 + jnp.exp(-z)))
    z = mm(w2_ref[...], z) + b2_ref[...]
    z = z * (1.0 / (1.0 + jnp.exp(-z)))
    heads = jnp.tanh(mm(wh_ref[...], z) + bh_ref[...])
    dchi = heads[0:4] * 0.5
    sraw = heads[4:18]

    iota20 = jax.lax.broadcasted_iota(jnp.int32, pr.shape, 0)
    mx = jnp.max(pr, axis=0, keepdims=True)
    idxf = jnp.min(jnp.where(pr == mx, iota20, 127), axis=0, keepdims=True)
    onehot = (iota20 == idxf).astype(f32)

    G = mm(meta_ref[...], onehot)

    X = [G[_R_TX:_R_TX + 14], G[_R_TY:_R_TY + 14], G[_R_TZ:_R_TZ + 14]]

    for gi in range(4):
        uw = G[_R_UW + 16 * gi:_R_UW + 16 * gi + 14]
        vw = G[_R_VW + 16 * gi:_R_VW + 16 * gi + 14]
        af = G[_R_AF + 16 * gi:_R_AF + 16 * gi + 14]
        pu = [jnp.sum(uw * X[c], axis=0, keepdims=True) for c in range(3)]
        pv = [jnp.sum(vw * X[c], axis=0, keepdims=True) for c in range(3)]
        ax = [pv[c] - pu[c] for c in range(3)]
        n2 = ax[0] * ax[0] + ax[1] * ax[1] + ax[2] * ax[2]
        inv = 1.0 / jnp.maximum(jnp.sqrt(n2), 1e-8)
        a = [ax[c] * inv for c in range(3)]
        ang = dchi[gi:gi + 1] * G[_R_AX + gi:_R_AX + gi + 1]
        ca = jnp.cos(ang)
        sa = jnp.sin(ang)
        V = [X[c] - pu[c] for c in range(3)]
        cr = [a[1] * V[2] - a[2] * V[1],
              a[2] * V[0] - a[0] * V[2],
              a[0] * V[1] - a[1] * V[0]]
        dt = a[0] * V[0] + a[1] * V[1] + a[2] * V[2]
        t1 = dt * (1.0 - ca)
        wm = af > 0.5
        X = [jnp.where(wm, V[c] * ca + cr[c] * sa + a[c] * t1 + pu[c], X[c])
             for c in range(3)]

    spf = G[_R_SP:_R_SP + 14]
    smf = G[_R_SM:_R_SM + 14]
    pp = [jnp.zeros_like(X[0]) for _ in range(3)]
    for k in range(14):
        m = spf == float(k)
        for c in range(3):
            pp[c] = jnp.where(m, X[c][k:k + 1], pp[c])
    sf = 1.0 + 0.1 * sraw
    wms = smf > 0.5
    X = [jnp.where(wms, pp[c] + (X[c] - pp[c]) * sf, X[c]) for c in range(3)]

    for i in range(3):
        out_ref[16 * i:16 * i + 14, :] = (
            rt[3 * i + 0:3 * i + 1] * X[0]
            + rt[3 * i + 1:3 * i + 2] * X[1]
            + rt[3 * i + 2:3 * i + 3] * X[2]
            + rt[9 + i:10 + i])
    out_ref[14:16, :] = jnp.zeros_like(out_ref[14:16, :])
    out_ref[30:32, :] = jnp.zeros_like(out_ref[30:32, :])
    out_ref[46:48, :] = jnp.zeros_like(out_ref[46:48, :])


def kernel(SH, aatype_probs, Rmats, tpos, W_e, b_e, W1, b1, W2, b2, W_t, b_t,
           W_s, b_s, template_local, template_exists, tors_axis, aff_idx,
           aff_mask, G_counts, scale_mask, scale_parent):
    f32 = jnp.float32
    B, N, DSH = SH.shape
    AAT = aatype_probs.shape[-1]
    H = W_e.shape[1]
    A14 = template_local.shape[1]
    GMAX = tors_axis.shape[1]
    BN = B * N

    shT = SH.reshape(BN, DSH).T
    prT = aatype_probs.reshape(BN, AAT).T
    rtT = jnp.concatenate([Rmats.reshape(BN, 9), tpos.reshape(BN, 3)], axis=1).T

    WeT = W_e.T
    W1hT = W1[:H].T
    W1pT = W1[H:].T
    W2T = W2.T
    WhT = jnp.concatenate([W_t, W_s], axis=1).T
    bh = jnp.concatenate([b_t, b_s])[:, None]
    beC = b_e[:, None]
    b1C = b1[:, None]
    b2C = b2[:, None]

    ex = template_exists.astype(f32)
    karr = jnp.arange(A14)
    u = tors_axis[..., 0]
    v = tors_axis[..., 1]
    uc = jnp.clip(u, 0, None)
    vc = jnp.clip(v, 0, None)
    uw = (uc[..., None] == karr).astype(f32)
    vw = (vc[..., None] == karr).astype(f32)
    gmask = (jnp.arange(GMAX)[None, :] < G_counts[:, None])
    axok = ((u >= 0) & (v >= 0) & gmask).astype(f32)
    aic = jnp.clip(aff_idx, 0, None)
    hit = aff_mask[..., None] & (aic[..., None] == karr)
    exg = jnp.take_along_axis(template_exists, aic.reshape(AAT, -1), axis=1)
    exg = exg.reshape(aic.shape)
    affm = jnp.any(hit & exg[..., None], axis=2).astype(f32) * ex[:, None, :]
    spc = jnp.clip(scale_parent, 0, None)
    spf = spc.astype(f32)
    exp_par = jnp.take_along_axis(template_exists, spc, axis=1)
    smf = (scale_mask & template_exists & exp_par).astype(f32)

    z2 = jnp.zeros((2, AAT), f32)
    secs = []
    for c in range(3):
        secs += [template_local[:, :, c].T, z2]
    for gi in range(GMAX):
        secs += [uw[:, gi, :].T, z2]
    for gi in range(GMAX):
        secs += [vw[:, gi, :].T, z2]
    for gi in range(GMAX):
        secs += [affm[:, gi, :].T, z2]
    secs += [spf.T, z2, smf.T, z2, axok.T,
             jnp.zeros((_ROWS - _R_AX - GMAX, AAT), f32)]
    meta = jnp.concatenate(secs, axis=0)

    grid = (BN // _T,)
    out48 = pl.pallas_call(
        _body,
        grid=grid,
        in_specs=[
            pl.BlockSpec((DSH, _T), lambda i: (0, i)),
            pl.BlockSpec((AAT, _T), lambda i: (0, i)),
            pl.BlockSpec((12, _T), lambda i: (0, i)),
            pl.BlockSpec((H, DSH), lambda i: (0, 0)),
            pl.BlockSpec((H, 1), lambda i: (0, 0)),
            pl.BlockSpec((H, H), lambda i: (0, 0)),
            pl.BlockSpec((H, AAT), lambda i: (0, 0)),
            pl.BlockSpec((H, 1), lambda i: (0, 0)),
            pl.BlockSpec((H, H), lambda i: (0, 0)),
            pl.BlockSpec((H, 1), lambda i: (0, 0)),
            pl.BlockSpec((18, H), lambda i: (0, 0)),
            pl.BlockSpec((18, 1), lambda i: (0, 0)),
            pl.BlockSpec((_ROWS, AAT), lambda i: (0, 0)),
        ],
        out_specs=pl.BlockSpec((48, _T), lambda i: (0, i)),
        out_shape=jax.ShapeDtypeStruct((48, BN), f32),
    )(shT, prT, rtT, WeT, beC, W1hT, W1pT, b1C, W2T, b2C, WhT, bh, meta)

    o = out48.reshape(3, 16, BN)[:, :A14, :]
    return jnp.transpose(o, (2, 1, 0)).reshape(B, N, A14, 3)
